# baseline (speedup 1.0000x reference)
import functools

import jax
import jax.numpy as jnp
from jax import lax
from jax.experimental import pallas as pl
from jax.experimental.pallas import tpu as pltpu
from jax.experimental.pallas import tpu_sc as plsc

_B, _N, _F, _E, _C = 4, 10000, 128, 160000, 10
_NP = 10240
_NC, _NS = 2, 16
_ET = _E // _NS
_NCH = 80
_CW = 64
_NCT = 160
_DR = _NP // _NS
_DGR = (4 * _NP) // _NS
_BN = 10000
_NG = _N // _BN
_KC = 51200
_KG = (_N * _F) // _KC

_mesh = plsc.VectorSubcoreMesh(core_axis_name="c", subcore_axis_name="s")


@functools.partial(
    pl.kernel,
    out_type=jax.ShapeDtypeStruct((_NC, 4 * _NP), jnp.float32),
    mesh=_mesh,
    scratch_types=[
        pltpu.VMEM_SHARED((4 * _NP,), jnp.float32),
        pltpu.VMEM((_NCH, 128), jnp.int32),
        pltpu.VMEM((128,), jnp.float32),
    ],
)
def _degree_kernel(didx_hbm, ones_hbm, z_hbm, out_hbm, accd, dv, onesv):
    c = lax.axis_index("c")
    s = lax.axis_index("s")

    @pl.when(s == 0)
    def _():
        pltpu.sync_copy(z_hbm, accd)
    pltpu.sync_copy(ones_hbm, onesv)
    plsc.subcore_barrier()
    for a in range(4):
        pltpu.sync_copy(didx_hbm.at[c, a, s], dv)

        def _chunk(j, carry):
            pltpu.sync_copy(onesv, accd.at[dv.at[j]], add=True)
            return carry

        lax.fori_loop(0, _NCH, _chunk, 0)
    plsc.subcore_barrier()

    @pl.when(s == 0)
    def _():
        pltpu.sync_copy(accd, out_hbm.at[c])


@functools.partial(
    pl.kernel,
    out_type=jax.ShapeDtypeStruct((_B, _NP, _F), jnp.float32),
    mesh=_mesh,
    scratch_types=[
        pltpu.VMEM_SHARED((_NP, _F), jnp.float32),
        pltpu.VMEM((_NCH // 2, 128), jnp.int32),
        pltpu.VMEM((_NCH // 2, 128), jnp.int32),
        pltpu.VMEM((128, _F), jnp.float32),
        pltpu.VMEM((128, _F), jnp.float32),
        pltpu.SemaphoreType.DMA,
        pltpu.SemaphoreType.DMA,
    ],
)
def _agg_kernel(hs_hbm, gidx_hbm, sidx_hbm, zrow_hbm, out_hbm,
                acc, gv, sv, rb0, rb1, sem0, sem1):
    c = lax.axis_index("c")
    s = lax.axis_index("s")
    nh = _NCH // 2
    for g in range(2):
        b = c * 2 + g
        pltpu.sync_copy(zrow_hbm, acc.at[pl.ds(s * _DR, _DR)])
        plsc.subcore_barrier()
        for ph in range(2):
            pltpu.sync_copy(gidx_hbm.at[b, s, pl.ds(ph * nh, nh)], gv)
            pltpu.sync_copy(sidx_hbm.at[b, s, pl.ds(ph * nh, nh)], sv)
            pltpu.async_copy(hs_hbm.at[gv.at[0]], rb0, sem0)
            pltpu.async_copy(hs_hbm.at[gv.at[1]], rb1, sem1)

            def _pair(jj, carry):
                j = 2 * jj
                pltpu.make_async_copy(hs_hbm.at[gv.at[j]], rb0, sem0).wait()
                pltpu.sync_copy(rb0, acc.at[sv.at[j]], add=True)
                pltpu.async_copy(hs_hbm.at[gv.at[j + 2]], rb0, sem0)
                pltpu.make_async_copy(hs_hbm.at[gv.at[j + 1]], rb1,
                                      sem1).wait()
                pltpu.sync_copy(rb1, acc.at[sv.at[j + 1]], add=True)
                pltpu.async_copy(hs_hbm.at[gv.at[j + 3]], rb1, sem1)
                return carry

            lax.fori_loop(0, nh // 2 - 2, _pair, 0)
            pltpu.make_async_copy(hs_hbm.at[gv.at[nh - 4]], rb0, sem0).wait()
            pltpu.sync_copy(rb0, acc.at[sv.at[nh - 4]], add=True)
            pltpu.async_copy(hs_hbm.at[gv.at[nh - 2]], rb0, sem0)
            pltpu.make_async_copy(hs_hbm.at[gv.at[nh - 3]], rb1, sem1).wait()
            pltpu.sync_copy(rb1, acc.at[sv.at[nh - 3]], add=True)
            pltpu.async_copy(hs_hbm.at[gv.at[nh - 1]], rb1, sem1)
            pltpu.make_async_copy(hs_hbm.at[gv.at[nh - 2]], rb0, sem0).wait()
            pltpu.sync_copy(rb0, acc.at[sv.at[nh - 2]], add=True)
            pltpu.make_async_copy(hs_hbm.at[gv.at[nh - 1]], rb1, sem1).wait()
            pltpu.sync_copy(rb1, acc.at[sv.at[nh - 1]], add=True)
        plsc.subcore_barrier()
        pltpu.sync_copy(acc.at[pl.ds(s * _DR, _DR)],
                        out_hbm.at[b, pl.ds(s * _DR, _DR)])
        plsc.subcore_barrier()


def _prep_body(feat_ref, deg_ref, hs_ref, nrm_ref):
    dg = deg_ref[0]
    nrm = jnp.where(dg > 0.0, lax.rsqrt(jnp.maximum(dg, 1.0)), 0.0)
    nrm_ref[0] = nrm
    hs_ref[0] = feat_ref[0] * nrm[:, 0:1]


def _layer1_body(p_ref, nrm_ref, w_ref, b_ref, out_ref):
    nrm = nrm_ref[0]
    agg = p_ref[0] * nrm[:, 1:2]
    h = jnp.dot(agg, w_ref[...], preferred_element_type=jnp.float32)
    h = jnp.maximum(h + b_ref[...], 0.0)
    out_ref[0] = h * nrm[:, 0:1]


def _layer2_body(p_ref, nrm_ref, w_ref, b_ref, x2_ref, st_ref):
    agg = p_ref[0] * nrm_ref[0][:, 1:2]
    x = jnp.dot(agg, w_ref[...], preferred_element_type=jnp.float32)
    x = x + b_ref[...]
    x2_ref[0] = x
    s1 = jnp.sum(x, axis=0)
    s2 = jnp.sum(x * x, axis=0)
    upd = jnp.concatenate(
        [s1[None, :], s2[None, :], jnp.zeros((6, _F), jnp.float32)], axis=0)
    first = jnp.logical_and(pl.program_id(0) == 0, pl.program_id(1) == 0)

    @pl.when(first)
    def _():
        st_ref[...] = upd

    @pl.when(jnp.logical_not(first))
    def _():
        st_ref[...] = st_ref[...] + upd


def _final_body(x2_ref, w_ref, a_ref, c_ref, out_ref):
    a5 = jnp.concatenate(
        [
            (x2_ref[...].reshape(_B, _KC // _F, _F)
             * a_ref[0][None, None, :]).reshape(_B, _KC),
            jnp.broadcast_to(c_ref[0][None, None, :],
                             (1, _KC // _F, _F)).reshape(1, _KC),
        ],
        axis=0,
    )
    m = lax.dot_general(a5, w_ref[...], (((1,), (0,)), ((), ())),
                        preferred_element_type=jnp.float32)
    m8 = jnp.concatenate(
        [m, jnp.zeros((8 - (_B + 1), _C), jnp.float32)], axis=0)
    upd = jnp.concatenate(
        [m8, jnp.zeros((8, 128 - _C), jnp.float32)], axis=1)

    @pl.when(pl.program_id(0) == 0)
    def _():
        out_ref[...] = upd

    @pl.when(pl.program_id(0) != 0)
    def _():
        out_ref[...] = out_ref[...] + upd


def kernel(features, edge_index, W1, b1, W2, b2, gamma, beta, W_lin, b_lin):
    f32, i32 = jnp.float32, jnp.int32
    src = edge_index[:, 0, :]
    dst = edge_index[:, 1, :]
    pad = _NCH * 128 - _ET

    boff = (jnp.arange(_B, dtype=i32) * _NP)[:, None, None]
    gsrc = src.reshape(_B, _NS, _ET) + boff
    gidx = jnp.concatenate(
        [gsrc, jnp.broadcast_to(boff, (_B, _NS, pad))], axis=-1
    ).reshape(_B, _NS, _NCH, 128)
    sidx = jnp.concatenate(
        [dst.reshape(_B, _NS, _ET),
         jnp.full((_B, _NS, pad), _N, i32)], axis=-1
    ).reshape(_B, _NS, _NCH, 128)

    ia = edge_index.reshape(_NC, 2, 2, _E).reshape(_NC, 4, _NS, _ET)
    aoff = (jnp.arange(4, dtype=i32) * _NP)[None, :, None, None]
    didx = jnp.concatenate(
        [ia + aoff, jnp.broadcast_to(aoff + _N, (_NC, 4, _NS, pad))],
        axis=-1,
    ).reshape(_NC, 4, _NS, _NCH, 128)

    ones128 = jnp.ones((128,), f32)
    zdeg = jnp.zeros((4 * _NP,), f32)
    zrow = jnp.zeros((_DR, _F), f32)

    degs = _degree_kernel(didx, ones128, zdeg)
    deg_t = degs.reshape(_NC, 2, 2, _NP).transpose(0, 1, 3, 2) \
                .reshape(_B, _NP, 2)
    hs0, nrm = pl.pallas_call(
        _prep_body,
        grid=(_B, _NG),
        in_specs=[pl.BlockSpec((1, _BN, _F), lambda b, n: (b, n, 0)),
                  pl.BlockSpec((1, _BN, 2), lambda b, n: (b, n, 0))],
        out_specs=[pl.BlockSpec((1, _BN, _F), lambda b, n: (b, n, 0)),
                   pl.BlockSpec((1, _BN, 2), lambda b, n: (b, n, 0))],
        out_shape=[jax.ShapeDtypeStruct((_B, _NP, _F), f32),
                   jax.ShapeDtypeStruct((_B, _NP, 2), f32)],
    )(features, deg_t)

    p1 = _agg_kernel(hs0.reshape(_B * _NP, _F), gidx, sidx, zrow)

    hs1 = pl.pallas_call(
        _layer1_body,
        grid=(_B, _NG),
        in_specs=[pl.BlockSpec((1, _BN, _F), lambda b, n: (b, n, 0)),
                  pl.BlockSpec((1, _BN, 2), lambda b, n: (b, n, 0)),
                  pl.BlockSpec((_F, _F), lambda b, n: (0, 0)),
                  pl.BlockSpec((1, _F), lambda b, n: (0, 0))],
        out_specs=pl.BlockSpec((1, _BN, _F), lambda b, n: (b, n, 0)),
        out_shape=jax.ShapeDtypeStruct((_B, _NP, _F), f32),
    )(p1, nrm, W1, b1.reshape(1, _F))

    p2 = _agg_kernel(hs1.reshape(_B * _NP, _F), gidx, sidx, zrow)

    x2, stats = pl.pallas_call(
        _layer2_body,
        grid=(_B, _NG),
        in_specs=[pl.BlockSpec((1, _BN, _F), lambda b, n: (b, n, 0)),
                  pl.BlockSpec((1, _BN, 2), lambda b, n: (b, n, 0)),
                  pl.BlockSpec((_F, _F), lambda b, n: (0, 0)),
                  pl.BlockSpec((1, _F), lambda b, n: (0, 0))],
        out_specs=[pl.BlockSpec((1, _BN, _F), lambda b, n: (b, n, 0)),
                   pl.BlockSpec((8, _F), lambda b, n: (0, 0))],
        out_shape=[jax.ShapeDtypeStruct((_B, _NP, _F), f32),
                   jax.ShapeDtypeStruct((8, _F), f32)],
    )(p2, nrm, W2, b2.reshape(1, _F))

    cnt = float(_B * _N)
    mean = stats[0, :] / cnt
    var = stats[1, :] / cnt - mean * mean
    avec = gamma * lax.rsqrt(var + 1e-5)
    cvec = beta - mean * avec

    macc = pl.pallas_call(
        _final_body,
        grid=(_KG,),
        in_specs=[pl.BlockSpec((_B, _KC), lambda k: (0, k)),
                  pl.BlockSpec((_KC, _C), lambda k: (k, 0)),
                  pl.BlockSpec((1, _F), lambda k: (0, 0)),
                  pl.BlockSpec((1, _F), lambda k: (0, 0))],
        out_specs=pl.BlockSpec((8, 128), lambda k: (0, 0)),
        out_shape=jax.ShapeDtypeStruct((8, 128), f32),
    )(x2.reshape(_B, _NP * _F), W_lin, avec.reshape(1, _F),
      cvec.reshape(1, _F))

    return macc[0:_B, 0:_C] + macc[_B:_B + 1, 0:_C] + b_lin[None, :]


# speedup vs baseline: 1.1257x; 1.1257x over previous
import functools

import jax
import jax.numpy as jnp
from jax import lax
from jax.experimental import pallas as pl
from jax.experimental.pallas import tpu as pltpu
from jax.experimental.pallas import tpu_sc as plsc

_B, _N, _F, _E, _C = 4, 10000, 128, 160000, 10
_NP = 10240
_NC, _NS = 2, 16
_ET = _E // _NS
_NCH = 80
_CW = 64
_NCT = 160
_DR = _NP // _NS
_DGR = (4 * _NP) // _NS
_BN = 10000
_NG = _N // _BN
_KG = 25
_KR = _N // _KG

_mesh = plsc.VectorSubcoreMesh(core_axis_name="c", subcore_axis_name="s")


@functools.partial(
    pl.kernel,
    out_type=jax.ShapeDtypeStruct((_NC, 4 * _NP), jnp.float32),
    mesh=_mesh,
    scratch_types=[
        pltpu.VMEM_SHARED((4 * _NP,), jnp.float32),
        pltpu.VMEM((_NCH, 128), jnp.int32),
        pltpu.VMEM((128,), jnp.float32),
    ],
)
def _degree_kernel(didx_hbm, ones_hbm, z_hbm, out_hbm, accd, dv, onesv):
    c = lax.axis_index("c")
    s = lax.axis_index("s")

    @pl.when(s == 0)
    def _():
        pltpu.sync_copy(z_hbm, accd)
    pltpu.sync_copy(ones_hbm, onesv)
    plsc.subcore_barrier()
    for a in range(4):
        pltpu.sync_copy(didx_hbm.at[c, a, s], dv)

        def _chunk(j, carry):
            pltpu.sync_copy(onesv, accd.at[dv.at[j]], add=True)
            return carry

        lax.fori_loop(0, _NCH, _chunk, 0)
    plsc.subcore_barrier()

    @pl.when(s == 0)
    def _():
        pltpu.sync_copy(accd, out_hbm.at[c])


@functools.partial(
    pl.kernel,
    out_type=jax.ShapeDtypeStruct((_B, _NP, _F), jnp.float32),
    mesh=_mesh,
    scratch_types=[
        pltpu.VMEM_SHARED((_NP, _F), jnp.float32),
        pltpu.VMEM((_NCH // 2, 128), jnp.int32),
        pltpu.VMEM((_NCH // 2, 128), jnp.int32),
        pltpu.VMEM((128, _F), jnp.float32),
        pltpu.VMEM((128, _F), jnp.float32),
        pltpu.SemaphoreType.DMA,
        pltpu.SemaphoreType.DMA,
    ],
)
def _agg_kernel(hs_hbm, gidx_hbm, sidx_hbm, zrow_hbm, out_hbm,
                acc, gv, sv, rb0, rb1, sem0, sem1):
    c = lax.axis_index("c")
    s = lax.axis_index("s")
    nh = _NCH // 2
    for g in range(2):
        b = c * 2 + g
        pltpu.sync_copy(zrow_hbm, acc.at[pl.ds(s * _DR, _DR)])
        plsc.subcore_barrier()
        for ph in range(2):
            pltpu.sync_copy(gidx_hbm.at[b, s, pl.ds(ph * nh, nh)], gv)
            pltpu.sync_copy(sidx_hbm.at[b, s, pl.ds(ph * nh, nh)], sv)
            pltpu.async_copy(hs_hbm.at[gv.at[0]], rb0, sem0)
            pltpu.async_copy(hs_hbm.at[gv.at[1]], rb1, sem1)

            def _pair(jj, carry):
                j = 2 * jj
                pltpu.make_async_copy(hs_hbm.at[gv.at[j]], rb0, sem0).wait()
                pltpu.sync_copy(rb0, acc.at[sv.at[j]], add=True)
                pltpu.async_copy(hs_hbm.at[gv.at[j + 2]], rb0, sem0)
                pltpu.make_async_copy(hs_hbm.at[gv.at[j + 1]], rb1,
                                      sem1).wait()
                pltpu.sync_copy(rb1, acc.at[sv.at[j + 1]], add=True)
                pltpu.async_copy(hs_hbm.at[gv.at[j + 3]], rb1, sem1)
                return carry

            lax.fori_loop(0, nh // 2 - 2, _pair, 0)
            pltpu.make_async_copy(hs_hbm.at[gv.at[nh - 4]], rb0, sem0).wait()
            pltpu.sync_copy(rb0, acc.at[sv.at[nh - 4]], add=True)
            pltpu.async_copy(hs_hbm.at[gv.at[nh - 2]], rb0, sem0)
            pltpu.make_async_copy(hs_hbm.at[gv.at[nh - 3]], rb1, sem1).wait()
            pltpu.sync_copy(rb1, acc.at[sv.at[nh - 3]], add=True)
            pltpu.async_copy(hs_hbm.at[gv.at[nh - 1]], rb1, sem1)
            pltpu.make_async_copy(hs_hbm.at[gv.at[nh - 2]], rb0, sem0).wait()
            pltpu.sync_copy(rb0, acc.at[sv.at[nh - 2]], add=True)
            pltpu.make_async_copy(hs_hbm.at[gv.at[nh - 1]], rb1, sem1).wait()
            pltpu.sync_copy(rb1, acc.at[sv.at[nh - 1]], add=True)
        plsc.subcore_barrier()
        pltpu.sync_copy(acc.at[pl.ds(s * _DR, _DR)],
                        out_hbm.at[b, pl.ds(s * _DR, _DR)])
        plsc.subcore_barrier()


def _prep_body(feat_ref, deg_ref, hs_ref, nrm_ref):
    dg = deg_ref[0]
    nrm = jnp.where(dg > 0.0, lax.rsqrt(jnp.maximum(dg, 1.0)), 0.0)
    nrm_ref[0] = nrm
    hs_ref[0] = feat_ref[0] * nrm[:, 0:1]


def _layer1_body(p_ref, nrm_ref, w_ref, b_ref, out_ref):
    nrm = nrm_ref[0]
    agg = p_ref[0] * nrm[:, 1:2]
    h = jnp.dot(agg, w_ref[...], preferred_element_type=jnp.float32)
    h = jnp.maximum(h + b_ref[...], 0.0)
    out_ref[0] = h * nrm[:, 0:1]


def _layer2_body(p_ref, nrm_ref, w_ref, b_ref, x2_ref, st_ref):
    agg = p_ref[0] * nrm_ref[0][:, 1:2]
    x = jnp.dot(agg, w_ref[...], preferred_element_type=jnp.float32)
    x = x + b_ref[...]
    x2_ref[0] = x
    s1 = jnp.sum(x, axis=0)
    s2 = jnp.sum(x * x, axis=0)
    upd = jnp.concatenate(
        [s1[None, :], s2[None, :], jnp.zeros((6, _F), jnp.float32)], axis=0)
    first = jnp.logical_and(pl.program_id(0) == 0, pl.program_id(1) == 0)

    @pl.when(first)
    def _():
        st_ref[...] = upd

    @pl.when(jnp.logical_not(first))
    def _():
        st_ref[...] = st_ref[...] + upd


def _final_body(x2_ref, w_ref, a_ref, c_ref, q_ref, out_ref):
    k = pl.program_id(0)
    wb = w_ref[...]
    qs = []
    for b in range(_B):
        qs.append(lax.dot_general(x2_ref[b], wb, (((0,), (0,)), ((), ())),
                                  preferred_element_type=jnp.float32))
    qs.append(lax.dot_general(jnp.ones((_KR, _F), jnp.float32), wb,
                              (((0,), (0,)), ((), ())),
                              preferred_element_type=jnp.float32))
    upd = jnp.stack(qs, axis=0)

    @pl.when(k == 0)
    def _():
        q_ref[...] = upd

    @pl.when(k != 0)
    def _():
        q_ref[...] = q_ref[...] + upd

    @pl.when(k == _KG - 1)
    def _():
        q = q_ref[...]
        fio = lax.broadcasted_iota(jnp.int32, (_F, _F * _C), 0)
        jio = lax.broadcasted_iota(jnp.int32, (_F, _F * _C), 1)
        acc = jnp.zeros((8, 128), jnp.float32)
        for b in range(_B):
            t = (q[b] * a_ref[0][:, None]
                 + q[_B] * c_ref[0][:, None])
            for c in range(_C):
                val = jnp.sum(jnp.where(jio == fio * _C + c, t, 0.0))
                bio = lax.broadcasted_iota(jnp.int32, (8, 128), 0)
                cio = lax.broadcasted_iota(jnp.int32, (8, 128), 1)
                acc = acc + jnp.where(
                    jnp.logical_and(bio == b, cio == c), val, 0.0)
        out_ref[...] = acc


def kernel(features, edge_index, W1, b1, W2, b2, gamma, beta, W_lin, b_lin):
    f32, i32 = jnp.float32, jnp.int32
    src = edge_index[:, 0, :]
    dst = edge_index[:, 1, :]
    pad = _NCH * 128 - _ET

    boff = (jnp.arange(_B, dtype=i32) * _NP)[:, None, None]
    gsrc = src.reshape(_B, _NS, _ET) + boff
    gidx = jnp.concatenate(
        [gsrc, jnp.broadcast_to(boff, (_B, _NS, pad))], axis=-1
    ).reshape(_B, _NS, _NCH, 128)
    sidx = jnp.concatenate(
        [dst.reshape(_B, _NS, _ET),
         jnp.full((_B, _NS, pad), _N, i32)], axis=-1
    ).reshape(_B, _NS, _NCH, 128)

    ia = edge_index.reshape(_NC, 2, 2, _E).reshape(_NC, 4, _NS, _ET)
    aoff = (jnp.arange(4, dtype=i32) * _NP)[None, :, None, None]
    didx = jnp.concatenate(
        [ia + aoff, jnp.broadcast_to(aoff + _N, (_NC, 4, _NS, pad))],
        axis=-1,
    ).reshape(_NC, 4, _NS, _NCH, 128)

    ones128 = jnp.ones((128,), f32)
    zdeg = jnp.zeros((4 * _NP,), f32)
    zrow = jnp.zeros((_DR, _F), f32)

    degs = _degree_kernel(didx, ones128, zdeg)
    deg_t = degs.reshape(_NC, 2, 2, _NP).transpose(0, 1, 3, 2) \
                .reshape(_B, _NP, 2)
    hs0, nrm = pl.pallas_call(
        _prep_body,
        grid=(_B, _NG),
        in_specs=[pl.BlockSpec((1, _BN, _F), lambda b, n: (b, n, 0)),
                  pl.BlockSpec((1, _BN, 2), lambda b, n: (b, n, 0))],
        out_specs=[pl.BlockSpec((1, _BN, _F), lambda b, n: (b, n, 0)),
                   pl.BlockSpec((1, _BN, 2), lambda b, n: (b, n, 0))],
        out_shape=[jax.ShapeDtypeStruct((_B, _NP, _F), f32),
                   jax.ShapeDtypeStruct((_B, _NP, 2), f32)],
    )(features, deg_t)

    p1 = _agg_kernel(hs0.reshape(_B * _NP, _F), gidx, sidx, zrow)

    hs1 = pl.pallas_call(
        _layer1_body,
        grid=(_B, _NG),
        in_specs=[pl.BlockSpec((1, _BN, _F), lambda b, n: (b, n, 0)),
                  pl.BlockSpec((1, _BN, 2), lambda b, n: (b, n, 0)),
                  pl.BlockSpec((_F, _F), lambda b, n: (0, 0)),
                  pl.BlockSpec((1, _F), lambda b, n: (0, 0))],
        out_specs=pl.BlockSpec((1, _BN, _F), lambda b, n: (b, n, 0)),
        out_shape=jax.ShapeDtypeStruct((_B, _NP, _F), f32),
    )(p1, nrm, W1, b1.reshape(1, _F))

    p2 = _agg_kernel(hs1.reshape(_B * _NP, _F), gidx, sidx, zrow)

    x2, stats = pl.pallas_call(
        _layer2_body,
        grid=(_B, _NG),
        in_specs=[pl.BlockSpec((1, _BN, _F), lambda b, n: (b, n, 0)),
                  pl.BlockSpec((1, _BN, 2), lambda b, n: (b, n, 0)),
                  pl.BlockSpec((_F, _F), lambda b, n: (0, 0)),
                  pl.BlockSpec((1, _F), lambda b, n: (0, 0))],
        out_specs=[pl.BlockSpec((1, _BN, _F), lambda b, n: (b, n, 0)),
                   pl.BlockSpec((8, _F), lambda b, n: (0, 0))],
        out_shape=[jax.ShapeDtypeStruct((_B, _NP, _F), f32),
                   jax.ShapeDtypeStruct((8, _F), f32)],
    )(p2, nrm, W2, b2.reshape(1, _F))

    cnt = float(_B * _N)
    mean = stats[0, :] / cnt
    var = stats[1, :] / cnt - mean * mean
    avec = gamma * lax.rsqrt(var + 1e-5)
    cvec = beta - mean * avec

    _, macc = pl.pallas_call(
        _final_body,
        grid=(_KG,),
        in_specs=[pl.BlockSpec((_B, _KR, _F), lambda k: (0, k, 0)),
                  pl.BlockSpec((_KR, _F * _C), lambda k: (k, 0)),
                  pl.BlockSpec((1, _F), lambda k: (0, 0)),
                  pl.BlockSpec((1, _F), lambda k: (0, 0))],
        out_specs=[pl.BlockSpec((_B + 1, _F, _F * _C),
                                lambda k: (0, 0, 0)),
                   pl.BlockSpec((8, 128), lambda k: (0, 0))],
        out_shape=[jax.ShapeDtypeStruct((_B + 1, _F, _F * _C), f32),
                   jax.ShapeDtypeStruct((8, 128), f32)],
    )(x2, W_lin.reshape(_N, _F * _C), avec.reshape(1, _F),
      cvec.reshape(1, _F))

    return macc[0:_B, 0:_C] + b_lin[None, :]
